# baseline (speedup 1.0000x reference)
import jax
import jax.numpy as jnp
from jax.experimental import pallas as pl


def _xt_kernel(x_ref, w_ref, b_ref, o_ref):
    o_ref[:] = (
        jnp.dot(x_ref[:], w_ref[:], preferred_element_type=jnp.float32) + b_ref[:]
    )


def _agg_kernel(a_ref, xt_ref, o_ref):
    o_ref[:] = jnp.dot(a_ref[:], xt_ref[:], preferred_element_type=jnp.float32)


def kernel(x, edge_index, W, b):
    B, n, d_in = x.shape
    d_out = W.shape[1]
    x2 = x.reshape(B * n, d_in)

    bm1 = 1000
    xt = pl.pallas_call(
        _xt_kernel,
        grid=(n // bm1,),
        in_specs=[
            pl.BlockSpec((bm1, d_in), lambda i: (i, 0)),
            pl.BlockSpec((d_in, d_out), lambda i: (0, 0)),
            pl.BlockSpec((1, d_out), lambda i: (0, 0)),
        ],
        out_specs=pl.BlockSpec((bm1, d_out), lambda i: (i, 0)),
        out_shape=jax.ShapeDtypeStruct((n, d_out), jnp.float32),
    )(x2, W, b.reshape(1, d_out))

    bm = 200
    out = pl.pallas_call(
        _agg_kernel,
        grid=(n // bm,),
        in_specs=[
            pl.BlockSpec((bm, n), lambda i: (i, 0)),
            pl.BlockSpec((n, d_out), lambda i: (0, 0)),
        ],
        out_specs=pl.BlockSpec((bm, d_out), lambda i: (i, 0)),
        out_shape=jax.ShapeDtypeStruct((n, d_out), jnp.float32),
    )(edge_index, xt)

    return out.reshape(B, n, d_out)


# speedup vs baseline: 1.0679x; 1.0679x over previous
import jax
import jax.numpy as jnp
from jax.experimental import pallas as pl
from jax.experimental.pallas import tpu as pltpu


def _fused_kernel(x_ref, w_ref, b_ref, a_ref, o_ref, xt_ref):
    @pl.when(pl.program_id(0) == 0)
    def _():
        xt_ref[:] = (
            jnp.dot(x_ref[:], w_ref[:], preferred_element_type=jnp.float32)
            + b_ref[:]
        )

    o_ref[:] = jnp.dot(a_ref[:], xt_ref[:], preferred_element_type=jnp.float32)


def kernel(x, edge_index, W, b):
    B, n, d_in = x.shape
    d_out = W.shape[1]
    x2 = x.reshape(B * n, d_in)

    bm = 200
    out = pl.pallas_call(
        _fused_kernel,
        grid=(n // bm,),
        in_specs=[
            pl.BlockSpec((n, d_in), lambda i: (0, 0)),
            pl.BlockSpec((d_in, d_out), lambda i: (0, 0)),
            pl.BlockSpec((1, d_out), lambda i: (0, 0)),
            pl.BlockSpec((bm, n), lambda i: (i, 0)),
        ],
        out_specs=pl.BlockSpec((bm, d_out), lambda i: (i, 0)),
        out_shape=jax.ShapeDtypeStruct((n, d_out), jnp.float32),
        scratch_shapes=[pltpu.VMEM((n, d_out), jnp.float32)],
    )(x2, W, b.reshape(1, d_out), edge_index)

    return out.reshape(B, n, d_out)


# speedup vs baseline: 1.0722x; 1.0040x over previous
import jax
import jax.numpy as jnp
from jax.experimental import pallas as pl
from jax.experimental.pallas import tpu as pltpu


def _fused_kernel(x_ref, w_ref, b_ref, a_ref, o_ref, xt_ref):
    @pl.when(pl.program_id(0) == 0)
    def _():
        xt_ref[:] = (
            jnp.dot(x_ref[:], w_ref[:], preferred_element_type=jnp.float32)
            + b_ref[:]
        )

    o_ref[:] = jnp.dot(a_ref[:], xt_ref[:], preferred_element_type=jnp.float32)


def kernel(x, edge_index, W, b):
    B, n, d_in = x.shape
    d_out = W.shape[1]
    x2 = x.reshape(B * n, d_in)

    bm = 400
    out = pl.pallas_call(
        _fused_kernel,
        grid=(n // bm,),
        in_specs=[
            pl.BlockSpec((n, d_in), lambda i: (0, 0)),
            pl.BlockSpec((d_in, d_out), lambda i: (0, 0)),
            pl.BlockSpec((1, d_out), lambda i: (0, 0)),
            pl.BlockSpec((bm, n), lambda i: (i, 0)),
        ],
        out_specs=pl.BlockSpec((bm, d_out), lambda i: (i, 0)),
        out_shape=jax.ShapeDtypeStruct((n, d_out), jnp.float32),
        scratch_shapes=[pltpu.VMEM((n, d_out), jnp.float32)],
    )(x2, W, b.reshape(1, d_out), edge_index)

    return out.reshape(B, n, d_out)
